# baseline (speedup 1.0000x reference)
import functools

import jax
import jax.numpy as jnp
from jax import lax
from jax.experimental import pallas as pl
from jax.experimental.pallas import tpu as pltpu
from jax.experimental.pallas import tpu_sc as plsc

COMMITMENT_COST = 0.25

ROWS_PER_BLOCK = 1024

_NC, _NS, _L = 2, 16, 16
_NW = _NC * _NS
_GCHUNK = 128


def _tc_argmin_kernel(xt_ref, e_ref, et_ref, qst_ref, idx_ref, loss_ref):
    xt = xt_ref[...]
    et = et_ref[...]

    xsq_t = jnp.sum(xt * xt, axis=0, keepdims=True)
    esq_t = jnp.sum(et * et, axis=1, keepdims=True)
    prod_t = lax.dot_general(
        et, xt, dimension_numbers=(((1,), (0,)), ((), ())),
        preferred_element_type=jnp.float32)
    dist_t = xsq_t + esq_t - 2.0 * prod_t

    idx = jnp.argmin(dist_t, axis=0).astype(jnp.int32)
    idx_ref[...] = idx.reshape(idx_ref.shape)

    onehot_t = (jax.lax.broadcasted_iota(jnp.int32, dist_t.shape, 0)
                == idx[None, :]).astype(jnp.float32)
    qst_ref[...] = lax.dot_general(
        e_ref[...], onehot_t, dimension_numbers=(((1,), (0,)), ((), ())),
        preferred_element_type=jnp.float32)

    m = jnp.min(dist_t, axis=0)
    loss_ref[...] = jnp.sum(m).reshape(1, 1, 1)


def _sc_gather_body(et_hbm, idx_hbm, q_hbm, idx_v, q_v, gsem):
    n_rows = idx_hbm.shape[0]
    bpw = n_rows // _NW
    wid = lax.axis_index("s") * _NC + lax.axis_index("c")
    base = wid * bpw
    pltpu.sync_copy(idx_hbm.at[pl.ds(base, bpw)], idx_v)
    for g in range(bpw // _GCHUNK):
        pltpu.async_copy(
            et_hbm.at[idx_v.at[pl.ds(g * _GCHUNK, _GCHUNK)]],
            q_v.at[pl.ds(g * _GCHUNK, _GCHUNK)], gsem)
    for g in range(bpw // _GCHUNK):
        pltpu.make_async_copy(
            et_hbm.at[idx_v.at[pl.ds(g * _GCHUNK, _GCHUNK)]],
            q_v.at[pl.ds(g * _GCHUNK, _GCHUNK)], gsem).wait()
    pltpu.sync_copy(q_v, q_hbm.at[pl.ds(base, bpw)])


@functools.partial(jax.jit, static_argnames=())
def kernel(inputs, embeddings):
    embedding_dim = embeddings.shape[0]
    num_embeddings = embeddings.shape[1]
    flat = inputs.reshape(-1, embedding_dim)
    n_rows = flat.shape[0]
    n_blocks = n_rows // ROWS_PER_BLOCK

    embeddings_t = embeddings.T
    flat_t = flat.T

    quantized, idx2d, loss_sum = pl.pallas_call(
        _tc_argmin_kernel,
        grid=(n_blocks,),
        in_specs=[
            pl.BlockSpec((embedding_dim, ROWS_PER_BLOCK), lambda i: (0, i)),
            pl.BlockSpec((embedding_dim, num_embeddings), lambda i: (0, 0)),
            pl.BlockSpec((num_embeddings, embedding_dim), lambda i: (0, 0)),
        ],
        out_specs=[
            pl.BlockSpec((embedding_dim, ROWS_PER_BLOCK), lambda i: (0, i)),
            pl.BlockSpec((1, 1, ROWS_PER_BLOCK), lambda i: (i, 0, 0)),
            pl.BlockSpec((1, 1, 1), lambda i: (i, 0, 0)),
        ],
        out_shape=[
            jax.ShapeDtypeStruct((embedding_dim, n_rows), jnp.float32),
            jax.ShapeDtypeStruct((n_blocks, 1, ROWS_PER_BLOCK), jnp.int32),
            jax.ShapeDtypeStruct((n_blocks, 1, 1), jnp.float32),
        ],
        compiler_params=pltpu.CompilerParams(
            dimension_semantics=("arbitrary",)),
    )(flat_t, embeddings, embeddings_t)
    encoding_indices = idx2d.reshape(n_rows)
    quantized = quantized.T

    quantized_st = quantized.reshape(inputs.shape)
    mean_sq = jnp.sum(loss_sum) / jnp.float32(inputs.size)
    commitment_loss = COMMITMENT_COST * mean_sq
    codebook_loss = mean_sq
    return (quantized_st, encoding_indices, commitment_loss, codebook_loss)


# speedup vs baseline: 1.0582x; 1.0582x over previous
import functools

import jax
import jax.numpy as jnp
from jax import lax
from jax.experimental import pallas as pl
from jax.experimental.pallas import tpu as pltpu
from jax.experimental.pallas import tpu_sc as plsc

COMMITMENT_COST = 0.25

ROWS_PER_BLOCK = 2048

_NC, _NS, _L = 2, 16, 16
_NW = _NC * _NS
_GCHUNK = 128


def _tc_argmin_kernel(xt_ref, e_ref, et_ref, qst_ref, idx_ref, loss_ref):
    xt = xt_ref[...]
    et = et_ref[...]

    xsq_t = jnp.sum(xt * xt, axis=0, keepdims=True)
    esq_t = jnp.sum(et * et, axis=1, keepdims=True)
    prod_t = lax.dot_general(
        et, xt, dimension_numbers=(((1,), (0,)), ((), ())),
        preferred_element_type=jnp.float32)
    dist_t = xsq_t + esq_t - 2.0 * prod_t

    idx = jnp.argmin(dist_t, axis=0).astype(jnp.int32)
    idx_ref[...] = idx.reshape(idx_ref.shape)

    onehot_t = (jax.lax.broadcasted_iota(jnp.int32, dist_t.shape, 0)
                == idx[None, :]).astype(jnp.float32)
    qst_ref[...] = lax.dot_general(
        e_ref[...], onehot_t, dimension_numbers=(((1,), (0,)), ((), ())),
        preferred_element_type=jnp.float32)

    m = jnp.min(dist_t, axis=0)
    loss_ref[...] = jnp.sum(m).reshape(1, 1, 1)


def _sc_gather_body(et_hbm, idx_hbm, q_hbm, idx_v, q_v, gsem):
    n_rows = idx_hbm.shape[0]
    bpw = n_rows // _NW
    wid = lax.axis_index("s") * _NC + lax.axis_index("c")
    base = wid * bpw
    pltpu.sync_copy(idx_hbm.at[pl.ds(base, bpw)], idx_v)
    for g in range(bpw // _GCHUNK):
        pltpu.async_copy(
            et_hbm.at[idx_v.at[pl.ds(g * _GCHUNK, _GCHUNK)]],
            q_v.at[pl.ds(g * _GCHUNK, _GCHUNK)], gsem)
    for g in range(bpw // _GCHUNK):
        pltpu.make_async_copy(
            et_hbm.at[idx_v.at[pl.ds(g * _GCHUNK, _GCHUNK)]],
            q_v.at[pl.ds(g * _GCHUNK, _GCHUNK)], gsem).wait()
    pltpu.sync_copy(q_v, q_hbm.at[pl.ds(base, bpw)])


@functools.partial(jax.jit, static_argnames=())
def kernel(inputs, embeddings):
    embedding_dim = embeddings.shape[0]
    num_embeddings = embeddings.shape[1]
    flat = inputs.reshape(-1, embedding_dim)
    n_rows = flat.shape[0]
    n_blocks = n_rows // ROWS_PER_BLOCK

    embeddings_t = embeddings.T
    flat_t = flat.T

    quantized, idx2d, loss_sum = pl.pallas_call(
        _tc_argmin_kernel,
        grid=(n_blocks,),
        in_specs=[
            pl.BlockSpec((embedding_dim, ROWS_PER_BLOCK), lambda i: (0, i)),
            pl.BlockSpec((embedding_dim, num_embeddings), lambda i: (0, 0)),
            pl.BlockSpec((num_embeddings, embedding_dim), lambda i: (0, 0)),
        ],
        out_specs=[
            pl.BlockSpec((embedding_dim, ROWS_PER_BLOCK), lambda i: (0, i)),
            pl.BlockSpec((1, 1, ROWS_PER_BLOCK), lambda i: (i, 0, 0)),
            pl.BlockSpec((1, 1, 1), lambda i: (i, 0, 0)),
        ],
        out_shape=[
            jax.ShapeDtypeStruct((embedding_dim, n_rows), jnp.float32),
            jax.ShapeDtypeStruct((n_blocks, 1, ROWS_PER_BLOCK), jnp.int32),
            jax.ShapeDtypeStruct((n_blocks, 1, 1), jnp.float32),
        ],
        compiler_params=pltpu.CompilerParams(
            dimension_semantics=("arbitrary",)),
    )(flat_t, embeddings, embeddings_t)
    encoding_indices = idx2d.reshape(n_rows)
    quantized = quantized.T

    quantized_st = quantized.reshape(inputs.shape)
    mean_sq = jnp.sum(loss_sum) / jnp.float32(inputs.size)
    commitment_loss = COMMITMENT_COST * mean_sq
    codebook_loss = mean_sq
    return (quantized_st, encoding_indices, commitment_loss, codebook_loss)


# speedup vs baseline: 1.0588x; 1.0005x over previous
import functools

import jax
import jax.numpy as jnp
from jax import lax
from jax.experimental import pallas as pl
from jax.experimental.pallas import tpu as pltpu

COMMITMENT_COST = 0.25

ROWS_PER_BLOCK = 2048


def _vq_block_kernel(xt_ref, e_ref, et_ref, qst_ref, idx_ref, loss_ref):
    xt = xt_ref[...]
    et = et_ref[...]

    xsq_t = jnp.sum(xt * xt, axis=0, keepdims=True)
    esq_t = jnp.sum(et * et, axis=1, keepdims=True)
    prod_t = lax.dot_general(
        et, xt, dimension_numbers=(((1,), (0,)), ((), ())),
        preferred_element_type=jnp.float32)
    dist_t = xsq_t + esq_t - 2.0 * prod_t

    idx = jnp.argmin(dist_t, axis=0).astype(jnp.int32)
    idx_ref[...] = idx.reshape(idx_ref.shape)

    onehot_t = (jax.lax.broadcasted_iota(jnp.int32, dist_t.shape, 0)
                == idx[None, :]).astype(jnp.float32)
    qst_ref[...] = lax.dot_general(
        e_ref[...], onehot_t, dimension_numbers=(((1,), (0,)), ((), ())),
        preferred_element_type=jnp.float32)

    m = jnp.min(dist_t, axis=0)
    loss_ref[...] = jnp.sum(m).reshape(1, 1, 1)


@functools.partial(jax.jit, static_argnames=())
def kernel(inputs, embeddings):
    embedding_dim = embeddings.shape[0]
    num_embeddings = embeddings.shape[1]
    flat = inputs.reshape(-1, embedding_dim)
    n_rows = flat.shape[0]
    n_blocks = n_rows // ROWS_PER_BLOCK

    embeddings_t = embeddings.T
    flat_t = flat.T

    quantized_t, idx2d, loss_sum = pl.pallas_call(
        _vq_block_kernel,
        grid=(n_blocks,),
        in_specs=[
            pl.BlockSpec((embedding_dim, ROWS_PER_BLOCK), lambda i: (0, i)),
            pl.BlockSpec((embedding_dim, num_embeddings), lambda i: (0, 0)),
            pl.BlockSpec((num_embeddings, embedding_dim), lambda i: (0, 0)),
        ],
        out_specs=[
            pl.BlockSpec((embedding_dim, ROWS_PER_BLOCK), lambda i: (0, i)),
            pl.BlockSpec((1, 1, ROWS_PER_BLOCK), lambda i: (i, 0, 0)),
            pl.BlockSpec((1, 1, 1), lambda i: (i, 0, 0)),
        ],
        out_shape=[
            jax.ShapeDtypeStruct((embedding_dim, n_rows), jnp.float32),
            jax.ShapeDtypeStruct((n_blocks, 1, ROWS_PER_BLOCK), jnp.int32),
            jax.ShapeDtypeStruct((n_blocks, 1, 1), jnp.float32),
        ],
        compiler_params=pltpu.CompilerParams(
            dimension_semantics=("arbitrary",)),
    )(flat_t, embeddings, embeddings_t)

    encoding_indices = idx2d.reshape(n_rows)
    quantized_st = quantized_t.T.reshape(inputs.shape)
    mean_sq = jnp.sum(loss_sum) / jnp.float32(inputs.size)
    commitment_loss = COMMITMENT_COST * mean_sq
    codebook_loss = mean_sq
    return (quantized_st, encoding_indices, commitment_loss, codebook_loss)
